# baseline (speedup 1.0000x reference)
import functools

import jax
import jax.numpy as jnp
from jax import lax
from jax.experimental import pallas as pl
from jax.experimental.pallas import tpu as pltpu
from jax.experimental.pallas import tpu_sc as plsc

BATCH = 16384
EMBED = 32
NC = 2
NS = 16
NW = NC * NS
B_PER_W = BATCH // NW
CHUNK = 128
K = B_PER_W // CHUNK
BN_EPS = 1e-5


def _sc_gather(u_idx, i_idx, user_table, movie_table):
    mesh = plsc.VectorSubcoreMesh(core_axis_name="c", subcore_axis_name="s")

    @functools.partial(
        pl.kernel,
        mesh=mesh,
        out_type=(
            jax.ShapeDtypeStruct((BATCH, EMBED), jnp.float32),
            jax.ShapeDtypeStruct((BATCH, EMBED), jnp.float32),
        ),
        scratch_types=[
            pltpu.VMEM((K, CHUNK), jnp.int32),
            pltpu.VMEM((K, CHUNK), jnp.int32),
            pltpu.VMEM((B_PER_W, EMBED), jnp.float32),
            pltpu.VMEM((B_PER_W, EMBED), jnp.float32),
            pltpu.SemaphoreType.DMA,
            pltpu.SemaphoreType.DMA,
        ],
    )
    def k(ut_hbm, mt_hbm, u_hbm, i_hbm, ou_hbm, oi_hbm,
          uidx_v, iidx_v, urows_v, irows_v, usem, isem):
        wid = lax.axis_index("s") * NC + lax.axis_index("c")
        base = wid * B_PER_W
        pltpu.sync_copy(u_hbm.at[wid], uidx_v)
        pltpu.sync_copy(i_hbm.at[wid], iidx_v)
        ucopies = []
        icopies = []
        for j in range(K):
            ucopies.append(pltpu.async_copy(
                ut_hbm.at[uidx_v.at[j]],
                urows_v.at[pl.ds(j * CHUNK, CHUNK)], usem))
            icopies.append(pltpu.async_copy(
                mt_hbm.at[iidx_v.at[j]],
                irows_v.at[pl.ds(j * CHUNK, CHUNK)], isem))
        for c in ucopies:
            c.wait()
        pltpu.sync_copy(urows_v, ou_hbm.at[pl.ds(base, B_PER_W)])
        for c in icopies:
            c.wait()
        pltpu.sync_copy(irows_v, oi_hbm.at[pl.ds(base, B_PER_W)])

    return k(user_table, movie_table, u_idx, i_idx)


def _mlp_body(xu_ref, xi_ref, g_ref, be_ref, mu_ref, var_ref,
              w1_ref, b1_ref, w2_ref, b2_ref, w3_ref, b3_ref,
              wo_ref, bo_ref, o_ref):
    scale = g_ref[...] * lax.rsqrt(var_ref[...] + BN_EPS)
    shift = be_ref[...] - mu_ref[...] * scale
    xu = xu_ref[...] * scale[:, :EMBED] + shift[:, :EMBED]
    xi = xi_ref[...] * scale[:, EMBED:] + shift[:, EMBED:]
    dot = functools.partial(jnp.dot, precision=lax.Precision.HIGHEST,
                            preferred_element_type=jnp.float32)
    h = dot(xu, w1_ref[:EMBED, :]) + dot(xi, w1_ref[EMBED:, :]) + b1_ref[...]
    h = jnp.maximum(h, 0.0)
    h = jnp.maximum(dot(h, w2_ref[...]) + b2_ref[...], 0.0)
    h = jnp.maximum(dot(h, w3_ref[...]) + b3_ref[...], 0.0)
    o_ref[...] = dot(h, wo_ref[...]) + bo_ref[...]


def _tc_mlp(xu, xi, bn_gamma, bn_beta, bn_mean, bn_var,
            W1, b1, W2, b2, W3, b3, Wo, bo):
    BLK = 4096
    grid = BATCH // BLK
    row_spec = pl.BlockSpec((BLK, EMBED), lambda i: (i, 0))

    def full(a):
        return pl.BlockSpec(a.shape, lambda i: (0,) * a.ndim)

    params = (bn_gamma, bn_beta, bn_mean, bn_var,
              W1, b1, W2, b2, W3, b3, Wo, bo)
    return pl.pallas_call(
        _mlp_body,
        grid=(grid,),
        in_specs=[row_spec, row_spec] + [full(p) for p in params],
        out_specs=pl.BlockSpec((BLK, 1), lambda i: (i, 0)),
        out_shape=jax.ShapeDtypeStruct((BATCH, 1), jnp.float32),
    )(xu, xi, *params)


def kernel(users, items, user_table, movie_table, bn_gamma, bn_beta,
           bn_mean, bn_var, W1, b1, W2, b2, W3, b3, Wo, bo):
    xu = jnp.take(user_table, users, axis=0)
    xi = jnp.take(movie_table, items, axis=0)
    two_d = lambda v: v.reshape(1, -1)
    return _tc_mlp(xu, xi, two_d(bn_gamma), two_d(bn_beta), two_d(bn_mean),
                   two_d(bn_var), W1, two_d(b1), W2, two_d(b2), W3,
                   two_d(b3), Wo, two_d(bo))
